# baseline (speedup 1.0000x reference)
import jax
import jax.numpy as jnp
from jax import lax
from jax.experimental import pallas as pl
from jax.experimental.pallas import tpu as pltpu
from jax.experimental.pallas import tpu_sc as plsc

NC = 2
NS = 16
NW = NC * NS
LANES = 16
C = 80
S = 2
ZR = 8


def _pre_kernel(h_ref, wsi_ref, wsa_ref, psi_ref, psa_ref):
    dn = (((1,), (1,)), ((), ()))
    hb = h_ref[...]
    psi_ref[...] = lax.dot_general(hb, wsi_ref[...], dn,
                                   preferred_element_type=jnp.float32)
    psa_ref[...] = lax.dot_general(hb, wsa_ref[...], dn,
                                   preferred_element_type=jnp.float32)


def _s8_kernel(h_ref, u8_ref, s8_ref):
    dn = (((1,), (1,)), ((), ()))
    s8_ref[...] = lax.dot_general(u8_ref[...], h_ref[...], dn,
                                  preferred_element_type=jnp.float32)


def _precompute(h, wsi, wsa, u8):
    n, d = h.shape
    b = 512
    psi, psa = pl.pallas_call(
        _pre_kernel,
        grid=(n // b,),
        in_specs=[
            pl.BlockSpec((b, d), lambda i: (i, 0)),
            pl.BlockSpec((d, d), lambda i: (0, 0)),
            pl.BlockSpec((d, d), lambda i: (0, 0)),
        ],
        out_specs=[
            pl.BlockSpec((b, d), lambda i: (i, 0)),
            pl.BlockSpec((b, d), lambda i: (i, 0)),
        ],
        out_shape=[
            jax.ShapeDtypeStruct((n, d), jnp.float32),
            jax.ShapeDtypeStruct((n, d), jnp.float32),
        ],
    )(h, wsi, wsa)
    s8 = pl.pallas_call(
        _s8_kernel,
        out_shape=jax.ShapeDtypeStruct((8, n), jnp.float32),
    )(h, u8)
    return psi, psa, s8


def _sc_reduce(p, sd, s12=None):
    n, d = p.shape
    nw, nch, two, c = sd.shape
    inter = s12 is not None
    assert nw == NW and c == C and two == 2 and nch % S == 0
    rpt = 640
    npad = ((n + NS * rpt - 1) // (NS * rpt)) * (NS * rpt)
    assert npad == n, "p must be pre-padded to the accumulator row count"

    scratch = [
        pltpu.VMEM((S, 2, C), jnp.int32),
        pltpu.VMEM((S, C, d), jnp.float32),
        pltpu.VMEM((S, C), jnp.float32),
        pltpu.VMEM((ZR, d), jnp.float32),
        pltpu.VMEM((rpt,), jnp.float32),
        pltpu.VMEM_SHARED((npad, d), jnp.float32),
        pltpu.VMEM_SHARED((npad,), jnp.float32),
    ]
    if inter:
        scratch += [pltpu.VMEM((n,), jnp.float32),
                    pltpu.VMEM((n,), jnp.float32)]
    scratch += [pltpu.SemaphoreType.DMA] * (4 * S)

    mesh = plsc.VectorSubcoreMesh(core_axis_name="c", subcore_axis_name="s")

    def body(*refs):
        if inter:
            (p_ref, sd_ref, s1_ref, s2_ref, acc_out, stat_out,
             sd_v, rows_v, ex_v, z2_v, z1_v, acc_sh, stat_sh,
             s1_v, s2_v, *sems) = refs
        else:
            (p_ref, sd_ref, acc_out, stat_out, sd_v, rows_v,
             ex_v, z2_v, z1_v, acc_sh, stat_sh, *sems) = refs
        sem_g = sems[0:S]
        sem_s = sems[S:2 * S]
        sem_t = sems[2 * S:3 * S]
        sem_i = sems[3 * S:4 * S]
        cid = lax.axis_index("c")
        sid = lax.axis_index("s")
        wid = cid * NS + sid
        zero16 = jnp.zeros((LANES,), jnp.float32)

        def zrow(r, _):
            for k in range(d // LANES):
                z2_v[r, pl.ds(k * LANES, LANES)] = zero16
            return 0

        lax.fori_loop(0, ZR, zrow, 0)

        def zflat(i, _):
            z1_v[pl.ds(i * LANES, LANES)] = zero16
            return 0

        lax.fori_loop(0, rpt // LANES, zflat, 0)
        for k in range(rpt // ZR):
            pltpu.sync_copy(z2_v, acc_sh.at[pl.ds(sid * rpt + k * ZR, ZR)])
        pltpu.sync_copy(z1_v, stat_sh.at[pl.ds(sid * rpt, rpt)])
        if inter:
            pltpu.sync_copy(s1_ref, s1_v)
            pltpu.sync_copy(s2_ref, s2_v)
        else:
            one16 = jnp.ones((LANES,), jnp.float32)
            for r in range(S):
                for gi in range(C // LANES):
                    ex_v[r, pl.ds(gi * LANES, LANES)] = one16
        plsc.subcore_barrier()

        dnums = lax.GatherDimensionNumbers(
            offset_dims=(), collapsed_slice_dims=(0,), start_index_map=(0,))

        def chunk(j, _):
            pltpu.sync_copy(sd_ref.at[wid, j], sd_v.at[0])
            pltpu.sync_copy(p_ref.at[sd_v.at[0, 0]], rows_v.at[0])
            if inter:
                exgs = []
                for gi in range(C // LANES):
                    sl = pl.ds(gi * LANES, LANES)
                    sv = sd_v[0, 0, sl]
                    dv = sd_v[0, 1, sl]
                    a = (plsc.load_gather(s1_v, [sv]) +
                         plsc.load_gather(s2_v, [dv]))
                    a = jnp.where(a >= 0.0, a, a * jnp.float32(0.01))
                    exg = jnp.exp(a)
                    ex_v[0, sl] = exg
                    exgs.append(exg)
                for gi in range(C // LANES):
                    for el in range(LANES):
                        spl = lax.gather(
                            exgs[gi], jnp.full((LANES, 1), el, jnp.int32),
                            dimension_numbers=dnums, slice_sizes=(1,),
                            mode=lax.GatherScatterMode.PROMISE_IN_BOUNDS)
                        eidx = gi * LANES + el
                        for k in range(d // LANES):
                            sl = pl.ds(k * LANES, LANES)
                            rows_v[0, eidx, sl] = rows_v[0, eidx, sl] * spl
            pltpu.sync_copy(rows_v.at[0], acc_sh.at[sd_v.at[0, 1]], add=True)
            pltpu.sync_copy(ex_v.at[0], stat_sh.at[sd_v.at[0, 1]], add=True)
            return 0

        lax.fori_loop(0, nch, chunk, 0)
        plsc.subcore_barrier()
        pltpu.sync_copy(acc_sh.at[pl.ds(sid * rpt, rpt)],
                        acc_out.at[cid, pl.ds(sid * rpt, rpt)])
        pltpu.sync_copy(stat_sh.at[pl.ds(sid * rpt, rpt)],
                        stat_out.at[cid, pl.ds(sid * rpt, rpt)])

    out_type = [jax.ShapeDtypeStruct((NC, npad, d), jnp.float32),
                jax.ShapeDtypeStruct((NC, npad), jnp.float32)]
    args = (p, sd, s12[0], s12[1]) if inter else (p, sd)
    return pl.kernel(
        body, out_type=out_type, mesh=mesh, scratch_types=scratch,
        compiler_params=pltpu.CompilerParams(needs_layout_passes=False),
    )(*args)


def _epi_kernel(acci_ref, cnt_ref, acca_ref, den_ref, h_ref, wdi_ref, wda_ref,
                wu1_ref, wu2_ref, cb_ref, o_ref):
    dn = (((1,), (1,)), ((), ()))
    hb = h_ref[...]
    ai = acci_ref[0] + acci_ref[1]
    aa = acca_ref[0] + acca_ref[1]
    cnt = (cnt_ref[0] + cnt_ref[1])[:, None]
    den = (den_ref[0] + den_ref[1])[:, None]
    ci = cb_ref[0:1, :]
    ca = cb_ref[1:2, :]
    bu = cb_ref[2:3, :]
    pdi = lax.dot_general(hb, wdi_ref[...], dn,
                          preferred_element_type=jnp.float32) + ci
    pda = lax.dot_general(hb, wda_ref[...], dn,
                          preferred_element_type=jnp.float32) + ca
    mi = ai / jnp.maximum(cnt, 1.0) + jnp.where(cnt > 0.0, pdi, 0.0)
    ma = aa / jnp.maximum(den, 1e-9) + jnp.where(den > 0.0, pda, 0.0)
    o = (lax.dot_general(mi, wu1_ref[...], dn,
                         preferred_element_type=jnp.float32) +
         lax.dot_general(ma, wu2_ref[...], dn,
                         preferred_element_type=jnp.float32) + bu)
    o_ref[...] = jnp.maximum(o, 0.0)


def _epilogue(acci, cnt, acca, den, h, wdi, wda, wu1, wu2, cb):
    n, d = h.shape
    npad = acci.shape[1]
    assert n == npad
    b = 512
    out = pl.pallas_call(
        _epi_kernel,
        grid=(npad // b,),
        in_specs=[
            pl.BlockSpec((NC, b, d), lambda i: (0, i, 0)),
            pl.BlockSpec((NC, b), lambda i: (0, i)),
            pl.BlockSpec((NC, b, d), lambda i: (0, i, 0)),
            pl.BlockSpec((NC, b), lambda i: (0, i)),
            pl.BlockSpec((b, d), lambda i: (i, 0)),
            pl.BlockSpec((d, d), lambda i: (0, 0)),
            pl.BlockSpec((d, d), lambda i: (0, 0)),
            pl.BlockSpec((d, d), lambda i: (0, 0)),
            pl.BlockSpec((d, d), lambda i: (0, 0)),
            pl.BlockSpec((8, d), lambda i: (0, 0)),
        ],
        out_specs=pl.BlockSpec((b, d), lambda i: (i, 0)),
        out_shape=jax.ShapeDtypeStruct((npad, d), jnp.float32),
    )(acci, cnt, acca, den, h, wdi, wda, wu1, wu2, cb)
    return out


def kernel(h, edge_index_intra, edge_index_inter, W_msg_intra, b_msg_intra,
           W_msg_inter, b_msg_inter, ef_intra, ef_inter, W_node_attn, W_attn,
           W_update, b_update):
    h = h.astype(jnp.float32)
    n, d = h.shape
    attn = W_node_attn.shape[0]
    wsi, wdi, wfi = (W_msg_intra[:, :d], W_msg_intra[:, d:2 * d],
                     W_msg_intra[:, 2 * d:])
    wsa, wda, wfa = (W_msg_inter[:, :d], W_msg_inter[:, d:2 * d],
                     W_msg_inter[:, 2 * d:])
    ci = wfi @ ef_intra + b_msg_intra
    ca = wfa @ ef_inter + b_msg_inter
    u1 = W_node_attn.T @ W_attn[0, :attn]
    u2 = W_node_attn.T @ W_attn[0, attn:]
    u8 = jnp.zeros((8, d), jnp.float32).at[0].set(u1).at[1].set(u2)
    cb = (jnp.zeros((8, d), jnp.float32)
          .at[0].set(ci).at[1].set(ca).at[2].set(b_update))
    wu1, wu2 = W_update[:, :d], W_update[:, d:]
    npad = NS * 640
    hp = jnp.pad(h, ((0, npad - n), (0, 0)))
    psi, psa, s8 = _precompute(hp, wsi, wsa, u8)

    def prep_edges(ei):
        ei = ei.astype(jnp.int32)
        e = ei.shape[1]
        ep = NW * npad
        pad = jnp.full((2, ep - e), n, jnp.int32)
        sd = jnp.concatenate([ei, pad], axis=1)
        return sd.reshape(2, NW, -1, C).transpose(1, 2, 0, 3)

    sd_i = prep_edges(edge_index_intra)
    sd_a = prep_edges(edge_index_inter)
    acci, cnt = _sc_reduce(psi, sd_i)
    acca, den = _sc_reduce(psa, sd_a, (s8[0], s8[1]))
    out = _epilogue(acci, cnt, acca, den, hp, wdi, wda, wu1, wu2, cb)
    return out[:n]


# speedup vs baseline: 1.6199x; 1.6199x over previous
import jax
import jax.numpy as jnp
from jax import lax
from jax.experimental import pallas as pl
from jax.experimental.pallas import tpu as pltpu
from jax.experimental.pallas import tpu_sc as plsc

NC = 2
NS = 16
NW = NC * NS
LANES = 16
C = 80
ZR = 40


def _pre_kernel(h_ref, wsi_ref, wsa_ref, psi_ref, psa_ref):
    dn = (((1,), (1,)), ((), ()))
    hb = h_ref[...]
    psi_ref[...] = lax.dot_general(hb, wsi_ref[...], dn,
                                   preferred_element_type=jnp.float32)
    psa_ref[...] = lax.dot_general(hb, wsa_ref[...], dn,
                                   preferred_element_type=jnp.float32)


def _s8_kernel(h_ref, u8_ref, s8_ref):
    dn = (((1,), (1,)), ((), ()))
    s8_ref[...] = lax.dot_general(u8_ref[...], h_ref[...], dn,
                                  preferred_element_type=jnp.float32)


def _precompute(h, wsi, wsa, u8):
    n, d = h.shape
    b = 400
    psi, psa = pl.pallas_call(
        _pre_kernel,
        grid=(n // b,),
        in_specs=[
            pl.BlockSpec((b, d), lambda i: (i, 0)),
            pl.BlockSpec((d, d), lambda i: (0, 0)),
            pl.BlockSpec((d, d), lambda i: (0, 0)),
        ],
        out_specs=[
            pl.BlockSpec((b, d), lambda i: (i, 0)),
            pl.BlockSpec((b, d), lambda i: (i, 0)),
        ],
        out_shape=[
            jax.ShapeDtypeStruct((n, d), jnp.float32),
            jax.ShapeDtypeStruct((n, d), jnp.float32),
        ],
    )(h, wsi, wsa)
    s8 = pl.pallas_call(
        _s8_kernel,
        out_shape=jax.ShapeDtypeStruct((8, n), jnp.float32),
    )(h, u8)
    return psi, psa, s8


def _sc_reduce(p, src, dst, s12=None):
    n, d = p.shape
    e = src.shape[0]
    inter = s12 is not None
    epw = e // NW
    nch = epw // C
    assert e == NW * nch * C and epw % 8 == 0
    g = C // LANES
    rpt = 640
    npad = ((n + NS * rpt - 1) // (NS * rpt)) * (NS * rpt)

    scratch = [
        pltpu.VMEM((C,), jnp.int32),
        pltpu.VMEM((C,), jnp.int32),
        pltpu.VMEM((C, d), jnp.float32),
        pltpu.VMEM((C,), jnp.float32),
        pltpu.VMEM((ZR, d), jnp.float32),
        pltpu.VMEM((rpt,), jnp.float32),
        pltpu.VMEM_SHARED((npad, d), jnp.float32),
        pltpu.VMEM_SHARED((npad,), jnp.float32),
    ]
    if inter:
        scratch += [pltpu.VMEM((n,), jnp.float32),
                    pltpu.VMEM((n,), jnp.float32)]

    mesh = plsc.VectorSubcoreMesh(core_axis_name="c", subcore_axis_name="s")

    def body(*refs):
        if inter:
            (p_ref, src_ref, dst_ref, s1_ref, s2_ref, acc_out, stat_out,
             src_v, dst_v, rows_v, ex_v, z2_v, z1_v, acc_sh, stat_sh,
             s1_v, s2_v) = refs
        else:
            (p_ref, src_ref, dst_ref, acc_out, stat_out, src_v, dst_v, rows_v,
             ex_v, z2_v, z1_v, acc_sh, stat_sh) = refs
        cid = lax.axis_index("c")
        sid = lax.axis_index("s")
        wid = cid * NS + sid
        zero16 = jnp.zeros((LANES,), jnp.float32)

        def zrow(r, _):
            for k in range(d // LANES):
                z2_v[r, pl.ds(k * LANES, LANES)] = zero16
            return 0

        lax.fori_loop(0, ZR, zrow, 0)

        def zflat(i, _):
            z1_v[pl.ds(i * LANES, LANES)] = zero16
            return 0

        lax.fori_loop(0, rpt // LANES, zflat, 0)
        for k in range(rpt // ZR):
            pltpu.sync_copy(z2_v, acc_sh.at[pl.ds(sid * rpt + k * ZR, ZR)])
        pltpu.sync_copy(z1_v, stat_sh.at[pl.ds(sid * rpt, rpt)])
        if inter:
            pltpu.sync_copy(s1_ref, s1_v)
            pltpu.sync_copy(s2_ref, s2_v)
        else:
            one16 = jnp.ones((LANES,), jnp.float32)
            for gi in range(g):
                ex_v[pl.ds(gi * LANES, LANES)] = one16
        plsc.subcore_barrier()

        dnums = lax.GatherDimensionNumbers(
            offset_dims=(), collapsed_slice_dims=(0,), start_index_map=(0,))

        def chunk(ch, _):
            base = wid * epw + ch * C
            pltpu.sync_copy(src_ref.at[pl.ds(base, C)], src_v)
            pltpu.sync_copy(dst_ref.at[pl.ds(base, C)], dst_v)
            pltpu.sync_copy(p_ref.at[src_v], rows_v)
            if inter:
                def grp(gi, _):
                    sv = src_v[pl.ds(gi * LANES, LANES)]
                    dv = dst_v[pl.ds(gi * LANES, LANES)]
                    a = (plsc.load_gather(s1_v, [sv]) +
                         plsc.load_gather(s2_v, [dv]))
                    a = jnp.where(a >= 0.0, a, a * jnp.float32(0.01))
                    exg = jnp.exp(a)
                    ex_v[pl.ds(gi * LANES, LANES)] = exg
                    for el in range(LANES):
                        spl = lax.gather(
                            exg, jnp.full((LANES, 1), el, jnp.int32),
                            dimension_numbers=dnums, slice_sizes=(1,),
                            mode=lax.GatherScatterMode.PROMISE_IN_BOUNDS)
                        eidx = gi * LANES + el
                        for k in range(d // LANES):
                            sl = pl.ds(k * LANES, LANES)
                            rows_v[eidx, sl] = rows_v[eidx, sl] * spl
                    return 0

                lax.fori_loop(0, g, grp, 0)
            pltpu.sync_copy(rows_v, acc_sh.at[dst_v], add=True)
            pltpu.sync_copy(ex_v, stat_sh.at[dst_v], add=True)
            return 0

        lax.fori_loop(0, nch, chunk, 0)
        plsc.subcore_barrier()
        pltpu.sync_copy(acc_sh.at[pl.ds(sid * rpt, rpt)],
                        acc_out.at[cid, pl.ds(sid * rpt, rpt)])
        pltpu.sync_copy(stat_sh.at[pl.ds(sid * rpt, rpt)],
                        stat_out.at[cid, pl.ds(sid * rpt, rpt)])

    out_type = [jax.ShapeDtypeStruct((NC, npad, d), jnp.float32),
                jax.ShapeDtypeStruct((NC, npad), jnp.float32)]
    args = (p, src, dst, s12[0], s12[1]) if inter else (p, src, dst)
    return pl.kernel(
        body, out_type=out_type, mesh=mesh, scratch_types=scratch,
        compiler_params=pltpu.CompilerParams(needs_layout_passes=False),
    )(*args)


def _epi_kernel(acci_ref, cnt_ref, acca_ref, den_ref, h_ref, wdi_ref, wda_ref,
                wu1_ref, wu2_ref, cb_ref, o_ref):
    dn = (((1,), (1,)), ((), ()))
    hb = h_ref[...]
    ai = acci_ref[0] + acci_ref[1]
    aa = acca_ref[0] + acca_ref[1]
    cnt = (cnt_ref[0] + cnt_ref[1])[:, None]
    den = (den_ref[0] + den_ref[1])[:, None]
    ci = cb_ref[0:1, :]
    ca = cb_ref[1:2, :]
    bu = cb_ref[2:3, :]
    pdi = lax.dot_general(hb, wdi_ref[...], dn,
                          preferred_element_type=jnp.float32) + ci
    pda = lax.dot_general(hb, wda_ref[...], dn,
                          preferred_element_type=jnp.float32) + ca
    mi = ai / jnp.maximum(cnt, 1.0) + jnp.where(cnt > 0.0, pdi, 0.0)
    ma = aa / jnp.maximum(den, 1e-9) + jnp.where(den > 0.0, pda, 0.0)
    o = (lax.dot_general(mi, wu1_ref[...], dn,
                         preferred_element_type=jnp.float32) +
         lax.dot_general(ma, wu2_ref[...], dn,
                         preferred_element_type=jnp.float32) + bu)
    o_ref[...] = jnp.maximum(o, 0.0)


def _epilogue(acci, cnt, acca, den, h, wdi, wda, wu1, wu2, cb):
    n, d = h.shape
    npad = acci.shape[1]
    b = 512
    hp = jnp.pad(h, ((0, npad - n), (0, 0)))
    out = pl.pallas_call(
        _epi_kernel,
        grid=(npad // b,),
        in_specs=[
            pl.BlockSpec((NC, b, d), lambda i: (0, i, 0)),
            pl.BlockSpec((NC, b), lambda i: (0, i)),
            pl.BlockSpec((NC, b, d), lambda i: (0, i, 0)),
            pl.BlockSpec((NC, b), lambda i: (0, i)),
            pl.BlockSpec((b, d), lambda i: (i, 0)),
            pl.BlockSpec((d, d), lambda i: (0, 0)),
            pl.BlockSpec((d, d), lambda i: (0, 0)),
            pl.BlockSpec((d, d), lambda i: (0, 0)),
            pl.BlockSpec((d, d), lambda i: (0, 0)),
            pl.BlockSpec((8, d), lambda i: (0, 0)),
        ],
        out_specs=pl.BlockSpec((b, d), lambda i: (i, 0)),
        out_shape=jax.ShapeDtypeStruct((npad, d), jnp.float32),
    )(acci, cnt, acca, den, hp, wdi, wda, wu1, wu2, cb)
    return out[:n]


def kernel(h, edge_index_intra, edge_index_inter, W_msg_intra, b_msg_intra,
           W_msg_inter, b_msg_inter, ef_intra, ef_inter, W_node_attn, W_attn,
           W_update, b_update):
    h = h.astype(jnp.float32)
    n, d = h.shape
    attn = W_node_attn.shape[0]
    wsi, wdi, wfi = (W_msg_intra[:, :d], W_msg_intra[:, d:2 * d],
                     W_msg_intra[:, 2 * d:])
    wsa, wda, wfa = (W_msg_inter[:, :d], W_msg_inter[:, d:2 * d],
                     W_msg_inter[:, 2 * d:])
    ci = wfi @ ef_intra + b_msg_intra
    ca = wfa @ ef_inter + b_msg_inter
    u1 = W_node_attn.T @ W_attn[0, :attn]
    u2 = W_node_attn.T @ W_attn[0, attn:]
    u8 = jnp.zeros((8, d), jnp.float32).at[0].set(u1).at[1].set(u2)
    cb = (jnp.zeros((8, d), jnp.float32)
          .at[0].set(ci).at[1].set(ca).at[2].set(b_update))
    wu1, wu2 = W_update[:, :d], W_update[:, d:]
    psi, psa, s8 = _precompute(h, wsi, wsa, u8)
    ei_i = edge_index_intra.astype(jnp.int32)
    ei_a = edge_index_inter.astype(jnp.int32)
    acci, cnt = _sc_reduce(psi, ei_i[0], ei_i[1])
    acca, den = _sc_reduce(psa, ei_a[0], ei_a[1], (s8[0], s8[1]))
    return _epilogue(acci, cnt, acca, den, h, wdi, wda, wu1, wu2, cb)


# speedup vs baseline: 1.6818x; 1.0382x over previous
import jax
import jax.numpy as jnp
from jax import lax
from jax.experimental import pallas as pl
from jax.experimental.pallas import tpu as pltpu
from jax.experimental.pallas import tpu_sc as plsc

NC = 2
NS = 16
NW = NC * NS
LANES = 16
C = 80
ZR = 40


def _pre_kernel(h_ref, wsi_ref, wsa_ref, psi_ref, psa_ref):
    dn = (((1,), (1,)), ((), ()))
    hb = h_ref[...]
    psi_ref[...] = lax.dot_general(hb, wsi_ref[...], dn,
                                   preferred_element_type=jnp.float32)
    psa_ref[...] = lax.dot_general(hb, wsa_ref[...], dn,
                                   preferred_element_type=jnp.float32)


def _s8_kernel(h_ref, u8_ref, s8_ref):
    dn = (((1,), (1,)), ((), ()))
    s8_ref[...] = lax.dot_general(u8_ref[...], h_ref[...], dn,
                                  preferred_element_type=jnp.float32)


def _precompute(h, wsi, wsa, u8):
    n, d = h.shape
    b = 400
    psi, psa = pl.pallas_call(
        _pre_kernel,
        grid=(n // b,),
        in_specs=[
            pl.BlockSpec((b, d), lambda i: (i, 0)),
            pl.BlockSpec((d, d), lambda i: (0, 0)),
            pl.BlockSpec((d, d), lambda i: (0, 0)),
        ],
        out_specs=[
            pl.BlockSpec((b, d), lambda i: (i, 0)),
            pl.BlockSpec((b, d), lambda i: (i, 0)),
        ],
        out_shape=[
            jax.ShapeDtypeStruct((n, d), jnp.float32),
            jax.ShapeDtypeStruct((n, d), jnp.float32),
        ],
    )(h, wsi, wsa)
    s8 = pl.pallas_call(
        _s8_kernel,
        out_shape=jax.ShapeDtypeStruct((8, n), jnp.float32),
    )(h, u8)
    return psi, psa, s8


def _sc_reduce(p, src, dst, s12=None):
    n, d = p.shape
    e = src.shape[0]
    inter = s12 is not None
    epw = e // NW
    nch = epw // C
    assert e == NW * nch * C and epw % 8 == 0
    g = C // LANES
    rpt = 640
    npad = ((n + NS * rpt - 1) // (NS * rpt)) * (NS * rpt)

    scratch = [
        pltpu.VMEM((C,), jnp.int32),
        pltpu.VMEM((C,), jnp.int32),
        pltpu.VMEM((C, d), jnp.float32),
        pltpu.VMEM((C,), jnp.float32),
        pltpu.VMEM((ZR, d), jnp.float32),
        pltpu.VMEM((rpt,), jnp.float32),
        pltpu.VMEM_SHARED((npad, d), jnp.float32),
        pltpu.VMEM_SHARED((npad,), jnp.float32),
    ]
    if inter:
        scratch += [pltpu.VMEM((n,), jnp.float32),
                    pltpu.VMEM((n,), jnp.float32)]
    scratch += [pltpu.SemaphoreType.DMA, pltpu.SemaphoreType.DMA]

    mesh = plsc.VectorSubcoreMesh(core_axis_name="c", subcore_axis_name="s")

    def body(*refs):
        if inter:
            (p_ref, src_ref, dst_ref, s1_ref, s2_ref, acc_out, stat_out,
             src_v, dst_v, rows_v, ex_v, z2_v, z1_v, acc_sh, stat_sh,
             s1_v, s2_v, sem_g, sem_t) = refs
        else:
            (p_ref, src_ref, dst_ref, acc_out, stat_out, src_v, dst_v, rows_v,
             ex_v, z2_v, z1_v, acc_sh, stat_sh, sem_g, sem_t) = refs
        cid = lax.axis_index("c")
        sid = lax.axis_index("s")
        wid = cid * NS + sid
        zero16 = jnp.zeros((LANES,), jnp.float32)

        def zrow(r, _):
            for k in range(d // LANES):
                z2_v[r, pl.ds(k * LANES, LANES)] = zero16
            return 0

        lax.fori_loop(0, ZR, zrow, 0)

        def zflat(i, _):
            z1_v[pl.ds(i * LANES, LANES)] = zero16
            return 0

        lax.fori_loop(0, rpt // LANES, zflat, 0)
        for k in range(rpt // ZR):
            pltpu.sync_copy(z2_v, acc_sh.at[pl.ds(sid * rpt + k * ZR, ZR)])
        pltpu.sync_copy(z1_v, stat_sh.at[pl.ds(sid * rpt, rpt)])
        if inter:
            pltpu.sync_copy(s1_ref, s1_v)
            pltpu.sync_copy(s2_ref, s2_v)
        else:
            one16 = jnp.ones((LANES,), jnp.float32)
            for gi in range(g):
                ex_v[pl.ds(gi * LANES, LANES)] = one16
        plsc.subcore_barrier()

        dnums = lax.GatherDimensionNumbers(
            offset_dims=(), collapsed_slice_dims=(0,), start_index_map=(0,))

        def chunk(ch, _):
            base = wid * epw + ch * C
            pltpu.sync_copy(src_ref.at[pl.ds(base, C)], src_v)
            pltpu.sync_copy(dst_ref.at[pl.ds(base, C)], dst_v)
            gat = pltpu.async_copy(p_ref.at[src_v], rows_v, sem_g)
            if inter:
                def exgrp(gi, _):
                    sv = src_v[pl.ds(gi * LANES, LANES)]
                    dv = dst_v[pl.ds(gi * LANES, LANES)]
                    a = (plsc.load_gather(s1_v, [sv]) +
                         plsc.load_gather(s2_v, [dv]))
                    a = jnp.where(a >= 0.0, a, a * jnp.float32(0.01))
                    ex_v[pl.ds(gi * LANES, LANES)] = jnp.exp(a)
                    return 0

                lax.fori_loop(0, g, exgrp, 0)
            gat.wait()
            pltpu.async_copy(ex_v, stat_sh.at[dst_v], sem_t, add=True)
            if inter:
                def scl(gi, _):
                    exg = ex_v[pl.ds(gi * LANES, LANES)]
                    for el in range(LANES):
                        spl = lax.gather(
                            exg, jnp.full((LANES, 1), el, jnp.int32),
                            dimension_numbers=dnums, slice_sizes=(1,),
                            mode=lax.GatherScatterMode.PROMISE_IN_BOUNDS)
                        eidx = gi * LANES + el
                        for k in range(d // LANES):
                            sl = pl.ds(k * LANES, LANES)
                            rows_v[eidx, sl] = rows_v[eidx, sl] * spl
                    return 0

                lax.fori_loop(0, g, scl, 0)
            pltpu.sync_copy(rows_v, acc_sh.at[dst_v], add=True)
            pltpu.make_async_copy(ex_v, stat_sh.at[dst_v], sem_t).wait()
            return 0

        lax.fori_loop(0, nch, chunk, 0)
        plsc.subcore_barrier()
        pltpu.sync_copy(acc_sh.at[pl.ds(sid * rpt, rpt)],
                        acc_out.at[cid, pl.ds(sid * rpt, rpt)])
        pltpu.sync_copy(stat_sh.at[pl.ds(sid * rpt, rpt)],
                        stat_out.at[cid, pl.ds(sid * rpt, rpt)])

    out_type = [jax.ShapeDtypeStruct((NC, npad, d), jnp.float32),
                jax.ShapeDtypeStruct((NC, npad), jnp.float32)]
    args = (p, src, dst, s12[0], s12[1]) if inter else (p, src, dst)
    return pl.kernel(
        body, out_type=out_type, mesh=mesh, scratch_types=scratch,
        compiler_params=pltpu.CompilerParams(needs_layout_passes=False),
    )(*args)


def _epi_kernel(acci_ref, cnt_ref, acca_ref, den_ref, h_ref, wdi_ref, wda_ref,
                wu1_ref, wu2_ref, cb_ref, o_ref):
    dn = (((1,), (1,)), ((), ()))
    hb = h_ref[...]
    ai = acci_ref[0] + acci_ref[1]
    aa = acca_ref[0] + acca_ref[1]
    cnt = (cnt_ref[0] + cnt_ref[1])[:, None]
    den = (den_ref[0] + den_ref[1])[:, None]
    ci = cb_ref[0:1, :]
    ca = cb_ref[1:2, :]
    bu = cb_ref[2:3, :]
    pdi = lax.dot_general(hb, wdi_ref[...], dn,
                          preferred_element_type=jnp.float32) + ci
    pda = lax.dot_general(hb, wda_ref[...], dn,
                          preferred_element_type=jnp.float32) + ca
    mi = ai / jnp.maximum(cnt, 1.0) + jnp.where(cnt > 0.0, pdi, 0.0)
    ma = aa / jnp.maximum(den, 1e-9) + jnp.where(den > 0.0, pda, 0.0)
    o = (lax.dot_general(mi, wu1_ref[...], dn,
                         preferred_element_type=jnp.float32) +
         lax.dot_general(ma, wu2_ref[...], dn,
                         preferred_element_type=jnp.float32) + bu)
    o_ref[...] = jnp.maximum(o, 0.0)


def _epilogue(acci, cnt, acca, den, h, wdi, wda, wu1, wu2, cb):
    n, d = h.shape
    npad = acci.shape[1]
    b = 512
    hp = jnp.pad(h, ((0, npad - n), (0, 0)))
    out = pl.pallas_call(
        _epi_kernel,
        grid=(npad // b,),
        in_specs=[
            pl.BlockSpec((NC, b, d), lambda i: (0, i, 0)),
            pl.BlockSpec((NC, b), lambda i: (0, i)),
            pl.BlockSpec((NC, b, d), lambda i: (0, i, 0)),
            pl.BlockSpec((NC, b), lambda i: (0, i)),
            pl.BlockSpec((b, d), lambda i: (i, 0)),
            pl.BlockSpec((d, d), lambda i: (0, 0)),
            pl.BlockSpec((d, d), lambda i: (0, 0)),
            pl.BlockSpec((d, d), lambda i: (0, 0)),
            pl.BlockSpec((d, d), lambda i: (0, 0)),
            pl.BlockSpec((8, d), lambda i: (0, 0)),
        ],
        out_specs=pl.BlockSpec((b, d), lambda i: (i, 0)),
        out_shape=jax.ShapeDtypeStruct((npad, d), jnp.float32),
    )(acci, cnt, acca, den, hp, wdi, wda, wu1, wu2, cb)
    return out[:n]


def kernel(h, edge_index_intra, edge_index_inter, W_msg_intra, b_msg_intra,
           W_msg_inter, b_msg_inter, ef_intra, ef_inter, W_node_attn, W_attn,
           W_update, b_update):
    h = h.astype(jnp.float32)
    n, d = h.shape
    attn = W_node_attn.shape[0]
    wsi, wdi, wfi = (W_msg_intra[:, :d], W_msg_intra[:, d:2 * d],
                     W_msg_intra[:, 2 * d:])
    wsa, wda, wfa = (W_msg_inter[:, :d], W_msg_inter[:, d:2 * d],
                     W_msg_inter[:, 2 * d:])
    ci = wfi @ ef_intra + b_msg_intra
    ca = wfa @ ef_inter + b_msg_inter
    u1 = W_node_attn.T @ W_attn[0, :attn]
    u2 = W_node_attn.T @ W_attn[0, attn:]
    u8 = jnp.zeros((8, d), jnp.float32).at[0].set(u1).at[1].set(u2)
    cb = (jnp.zeros((8, d), jnp.float32)
          .at[0].set(ci).at[1].set(ca).at[2].set(b_update))
    wu1, wu2 = W_update[:, :d], W_update[:, d:]
    psi, psa, s8 = _precompute(h, wsi, wsa, u8)
    ei_i = edge_index_intra.astype(jnp.int32)
    ei_a = edge_index_inter.astype(jnp.int32)
    acci, cnt = _sc_reduce(psi, ei_i[0], ei_i[1])
    acca, den = _sc_reduce(psa, ei_a[0], ei_a[1], (s8[0], s8[1]))
    return _epilogue(acci, cnt, acca, den, h, wdi, wda, wu1, wu2, cb)


# speedup vs baseline: 1.9496x; 1.1593x over previous
import jax
import jax.numpy as jnp
from jax import lax
from jax.experimental import pallas as pl
from jax.experimental.pallas import tpu as pltpu
from jax.experimental.pallas import tpu_sc as plsc

NC = 2
NS = 16
NW = NC * NS
LANES = 16
C = 80
ZR = 40


def _pre_kernel(h_ref, wsi_ref, wsa_ref, psi_ref, psa_ref):
    dn = (((1,), (1,)), ((), ()))
    hb = h_ref[...]
    psi_ref[...] = lax.dot_general(hb, wsi_ref[...], dn,
                                   preferred_element_type=jnp.float32)
    psa_ref[...] = lax.dot_general(hb, wsa_ref[...], dn,
                                   preferred_element_type=jnp.float32)


def _s8_kernel(h_ref, u8_ref, s8_ref):
    dn = (((1,), (1,)), ((), ()))
    s8_ref[...] = lax.dot_general(u8_ref[...], h_ref[...], dn,
                                  preferred_element_type=jnp.float32)


def _precompute(h, wsi, wsa, u8):
    n, d = h.shape
    b = 400
    psi, psa = pl.pallas_call(
        _pre_kernel,
        grid=(n // b,),
        in_specs=[
            pl.BlockSpec((b, d), lambda i: (i, 0)),
            pl.BlockSpec((d, d), lambda i: (0, 0)),
            pl.BlockSpec((d, d), lambda i: (0, 0)),
        ],
        out_specs=[
            pl.BlockSpec((b, d), lambda i: (i, 0)),
            pl.BlockSpec((b, d), lambda i: (i, 0)),
        ],
        out_shape=[
            jax.ShapeDtypeStruct((n, d), jnp.float32),
            jax.ShapeDtypeStruct((n, d), jnp.float32),
        ],
    )(h, wsi, wsa)
    s8 = pl.pallas_call(
        _s8_kernel,
        out_shape=jax.ShapeDtypeStruct((8, n), jnp.float32),
    )(h, u8)
    return psi, psa, s8


def _sc_reduce(p, src, dst, s12=None):
    n, d = p.shape
    e = src.shape[0]
    inter = s12 is not None
    epw = e // NW
    nch = epw // C
    assert e == NW * nch * C and epw % 8 == 0
    g = C // LANES
    rpt = 640
    npad = ((n + NS * rpt - 1) // (NS * rpt)) * (NS * rpt)

    scratch = [
        pltpu.VMEM((C,), jnp.int32),
        pltpu.VMEM((C,), jnp.int32),
        pltpu.VMEM((C, d), jnp.float32),
        pltpu.VMEM((C,), jnp.float32),
        pltpu.VMEM((C,), jnp.int32),
        pltpu.VMEM((C,), jnp.int32),
        pltpu.VMEM((C, d), jnp.float32),
        pltpu.VMEM((C,), jnp.float32),
        pltpu.VMEM((ZR, d), jnp.float32),
        pltpu.VMEM((rpt,), jnp.float32),
        pltpu.VMEM_SHARED((npad, d), jnp.float32),
        pltpu.VMEM_SHARED((npad,), jnp.float32),
    ]
    if inter:
        scratch += [pltpu.VMEM((n,), jnp.float32),
                    pltpu.VMEM((n,), jnp.float32)]
    scratch += [pltpu.SemaphoreType.DMA] * 6

    mesh = plsc.VectorSubcoreMesh(core_axis_name="c", subcore_axis_name="s")

    def body(*refs):
        if inter:
            (p_ref, src_ref, dst_ref, s1_ref, s2_ref, acc_out, stat_out,
             src_a, dst_a, rows_a, ex_a, src_b, dst_b, rows_b, ex_b,
             z2_v, z1_v, acc_sh, stat_sh, s1_v, s2_v,
             sem_ga, sem_ta, sem_sa, sem_gb, sem_tb, sem_sb) = refs
        else:
            (p_ref, src_ref, dst_ref, acc_out, stat_out,
             src_a, dst_a, rows_a, ex_a, src_b, dst_b, rows_b, ex_b,
             z2_v, z1_v, acc_sh, stat_sh,
             sem_ga, sem_ta, sem_sa, sem_gb, sem_tb, sem_sb) = refs
        cid = lax.axis_index("c")
        sid = lax.axis_index("s")
        wid = cid * NS + sid
        zero16 = jnp.zeros((LANES,), jnp.float32)

        def zrow(r, _):
            for k in range(d // LANES):
                z2_v[r, pl.ds(k * LANES, LANES)] = zero16
            return 0

        lax.fori_loop(0, ZR, zrow, 0)

        def zflat(i, _):
            z1_v[pl.ds(i * LANES, LANES)] = zero16
            return 0

        lax.fori_loop(0, rpt // LANES, zflat, 0)
        for k in range(rpt // ZR):
            pltpu.sync_copy(z2_v, acc_sh.at[pl.ds(sid * rpt + k * ZR, ZR)])
        pltpu.sync_copy(z1_v, stat_sh.at[pl.ds(sid * rpt, rpt)])
        if inter:
            pltpu.sync_copy(s1_ref, s1_v)
            pltpu.sync_copy(s2_ref, s2_v)
        else:
            one16 = jnp.ones((LANES,), jnp.float32)
            for gi in range(g):
                ex_a[pl.ds(gi * LANES, LANES)] = one16
                ex_b[pl.ds(gi * LANES, LANES)] = one16
        plsc.subcore_barrier()

        dnums = lax.GatherDimensionNumbers(
            offset_dims=(), collapsed_slice_dims=(0,), start_index_map=(0,))

        def process(ch, i, src_v, dst_v, rows_v, ex_v, sem_g, sem_t, sem_s):
            def drains():
                pltpu.make_async_copy(rows_v, acc_sh.at[dst_v], sem_s).wait()
                pltpu.make_async_copy(ex_v, stat_sh.at[dst_v], sem_t).wait()

            if i is None:
                drains()
            else:
                pl.when(i >= 1)(drains)
            base = wid * epw + ch * C
            pltpu.sync_copy(src_ref.at[pl.ds(base, C)], src_v)
            pltpu.sync_copy(dst_ref.at[pl.ds(base, C)], dst_v)
            gat = pltpu.async_copy(p_ref.at[src_v], rows_v, sem_g)
            if inter:
                def exgrp(gi, _):
                    sv = src_v[pl.ds(gi * LANES, LANES)]
                    dv = dst_v[pl.ds(gi * LANES, LANES)]
                    a = (plsc.load_gather(s1_v, [sv]) +
                         plsc.load_gather(s2_v, [dv]))
                    a = jnp.where(a >= 0.0, a, a * jnp.float32(0.01))
                    ex_v[pl.ds(gi * LANES, LANES)] = jnp.exp(a)
                    return 0

                lax.fori_loop(0, g, exgrp, 0)
            gat.wait()
            pltpu.async_copy(ex_v, stat_sh.at[dst_v], sem_t, add=True)
            if inter:
                def scl(gi, _):
                    exg = ex_v[pl.ds(gi * LANES, LANES)]
                    for el in range(LANES):
                        spl = lax.gather(
                            exg, jnp.full((LANES, 1), el, jnp.int32),
                            dimension_numbers=dnums, slice_sizes=(1,),
                            mode=lax.GatherScatterMode.PROMISE_IN_BOUNDS)
                        eidx = gi * LANES + el
                        for k in range(d // LANES):
                            sl = pl.ds(k * LANES, LANES)
                            rows_v[eidx, sl] = rows_v[eidx, sl] * spl
                    return 0

                lax.fori_loop(0, g, scl, 0)
            pltpu.async_copy(rows_v, acc_sh.at[dst_v], sem_s, add=True)

        bufs_a = (src_a, dst_a, rows_a, ex_a, sem_ga, sem_ta, sem_sa)
        bufs_b = (src_b, dst_b, rows_b, ex_b, sem_gb, sem_tb, sem_sb)

        def pair(i, _):
            process(2 * i, i, *bufs_a)
            process(2 * i + 1, i, *bufs_b)
            return 0

        lax.fori_loop(0, nch // 2, pair, 0)
        if nch % 2 == 1:
            process(nch - 1, None, *bufs_a)
            pltpu.make_async_copy(rows_a, acc_sh.at[dst_a], sem_sa).wait()
            pltpu.make_async_copy(ex_a, stat_sh.at[dst_a], sem_ta).wait()
        else:
            pltpu.make_async_copy(rows_a, acc_sh.at[dst_a], sem_sa).wait()
            pltpu.make_async_copy(ex_a, stat_sh.at[dst_a], sem_ta).wait()
        pltpu.make_async_copy(rows_b, acc_sh.at[dst_b], sem_sb).wait()
        pltpu.make_async_copy(ex_b, stat_sh.at[dst_b], sem_tb).wait()
        plsc.subcore_barrier()
        pltpu.sync_copy(acc_sh.at[pl.ds(sid * rpt, rpt)],
                        acc_out.at[cid, pl.ds(sid * rpt, rpt)])
        pltpu.sync_copy(stat_sh.at[pl.ds(sid * rpt, rpt)],
                        stat_out.at[cid, pl.ds(sid * rpt, rpt)])

    out_type = [jax.ShapeDtypeStruct((NC, npad, d), jnp.float32),
                jax.ShapeDtypeStruct((NC, npad), jnp.float32)]
    args = (p, src, dst, s12[0], s12[1]) if inter else (p, src, dst)
    return pl.kernel(
        body, out_type=out_type, mesh=mesh, scratch_types=scratch,
        compiler_params=pltpu.CompilerParams(needs_layout_passes=False),
    )(*args)


def _epi_kernel(acci_ref, cnt_ref, acca_ref, den_ref, h_ref, wdi_ref, wda_ref,
                wu1_ref, wu2_ref, cb_ref, o_ref):
    dn = (((1,), (1,)), ((), ()))
    hb = h_ref[...]
    ai = acci_ref[0] + acci_ref[1]
    aa = acca_ref[0] + acca_ref[1]
    cnt = (cnt_ref[0] + cnt_ref[1])[:, None]
    den = (den_ref[0] + den_ref[1])[:, None]
    ci = cb_ref[0:1, :]
    ca = cb_ref[1:2, :]
    bu = cb_ref[2:3, :]
    pdi = lax.dot_general(hb, wdi_ref[...], dn,
                          preferred_element_type=jnp.float32) + ci
    pda = lax.dot_general(hb, wda_ref[...], dn,
                          preferred_element_type=jnp.float32) + ca
    mi = ai / jnp.maximum(cnt, 1.0) + jnp.where(cnt > 0.0, pdi, 0.0)
    ma = aa / jnp.maximum(den, 1e-9) + jnp.where(den > 0.0, pda, 0.0)
    o = (lax.dot_general(mi, wu1_ref[...], dn,
                         preferred_element_type=jnp.float32) +
         lax.dot_general(ma, wu2_ref[...], dn,
                         preferred_element_type=jnp.float32) + bu)
    o_ref[...] = jnp.maximum(o, 0.0)


def _epilogue(acci, cnt, acca, den, h, wdi, wda, wu1, wu2, cb):
    n, d = h.shape
    npad = acci.shape[1]
    b = 512
    hp = jnp.pad(h, ((0, npad - n), (0, 0)))
    out = pl.pallas_call(
        _epi_kernel,
        grid=(npad // b,),
        in_specs=[
            pl.BlockSpec((NC, b, d), lambda i: (0, i, 0)),
            pl.BlockSpec((NC, b), lambda i: (0, i)),
            pl.BlockSpec((NC, b, d), lambda i: (0, i, 0)),
            pl.BlockSpec((NC, b), lambda i: (0, i)),
            pl.BlockSpec((b, d), lambda i: (i, 0)),
            pl.BlockSpec((d, d), lambda i: (0, 0)),
            pl.BlockSpec((d, d), lambda i: (0, 0)),
            pl.BlockSpec((d, d), lambda i: (0, 0)),
            pl.BlockSpec((d, d), lambda i: (0, 0)),
            pl.BlockSpec((8, d), lambda i: (0, 0)),
        ],
        out_specs=pl.BlockSpec((b, d), lambda i: (i, 0)),
        out_shape=jax.ShapeDtypeStruct((npad, d), jnp.float32),
    )(acci, cnt, acca, den, hp, wdi, wda, wu1, wu2, cb)
    return out[:n]


def kernel(h, edge_index_intra, edge_index_inter, W_msg_intra, b_msg_intra,
           W_msg_inter, b_msg_inter, ef_intra, ef_inter, W_node_attn, W_attn,
           W_update, b_update):
    h = h.astype(jnp.float32)
    n, d = h.shape
    attn = W_node_attn.shape[0]
    wsi, wdi, wfi = (W_msg_intra[:, :d], W_msg_intra[:, d:2 * d],
                     W_msg_intra[:, 2 * d:])
    wsa, wda, wfa = (W_msg_inter[:, :d], W_msg_inter[:, d:2 * d],
                     W_msg_inter[:, 2 * d:])
    ci = wfi @ ef_intra + b_msg_intra
    ca = wfa @ ef_inter + b_msg_inter
    u1 = W_node_attn.T @ W_attn[0, :attn]
    u2 = W_node_attn.T @ W_attn[0, attn:]
    u8 = jnp.zeros((8, d), jnp.float32).at[0].set(u1).at[1].set(u2)
    cb = (jnp.zeros((8, d), jnp.float32)
          .at[0].set(ci).at[1].set(ca).at[2].set(b_update))
    wu1, wu2 = W_update[:, :d], W_update[:, d:]
    psi, psa, s8 = _precompute(h, wsi, wsa, u8)
    ei_i = edge_index_intra.astype(jnp.int32)
    ei_a = edge_index_inter.astype(jnp.int32)
    acci, cnt = _sc_reduce(psi, ei_i[0], ei_i[1])
    acca, den = _sc_reduce(psa, ei_a[0], ei_a[1], (s8[0], s8[1]))
    return _epilogue(acci, cnt, acca, den, h, wdi, wda, wu1, wu2, cb)
